# Initial kernel scaffold; baseline (speedup 1.0000x reference)
import jax
import jax.numpy as jnp
from jax import lax
from jax.experimental import pallas as pl
from jax.experimental.pallas import tpu as pltpu
from jax.experimental.pallas import tpu_sc as plsc

N_IDENT = 10000
N_SYM = 10000
N_AST = 100000
N_OCC = 320000
D = 128

NC, NS = 2, 16
NW = NC * NS
OCC_W = N_OCC // NW
CHUNK = 80
NCHUNK = OCC_W // CHUNK
SYM_PAD = 10240
SYM_W = SYM_PAD // NW
SYM_CHUNKS = SYM_W // CHUNK
ROWS_T = N_SYM // NS
CP = 125
NCP = ROWS_T // CP


def _sc_gather_segsum(ident_tab, sym_idx, ast_tab, node_idx, seg_idx):
  mesh = plsc.VectorSubcoreMesh(
      core_axis_name="c", subcore_axis_name="s", num_cores=NC, num_subcores=NS)

  def body(ident_hbm, sym_hbm, ast_hbm, nidx_hbm, sidx_hbm, a_out, b_out,
           nidx_v, segv, symv, rows, zbuf, acc, sem):
    c = lax.axis_index("c")
    s = lax.axis_index("s")
    wid = s * NC + c

    pltpu.sync_copy(nidx_hbm.at[wid], nidx_v)
    pltpu.sync_copy(sidx_hbm.at[wid], segv)
    pltpu.sync_copy(sym_hbm.at[wid], symv)

    for k in range(SYM_CHUNKS):
      pltpu.async_copy(ident_hbm.at[symv.at[k]], rows, sem).wait()
      pltpu.sync_copy(rows, a_out.at[wid, pl.ds(k * CHUNK, CHUNK)])

    zero = jnp.zeros((16,), jnp.float32)

    @pl.loop(0, CP)
    def _zero_rows(i):
      for j in range(D // 16):
        zbuf[i, pl.ds(j * 16, 16)] = zero

    for m in range(NCP):
      pltpu.sync_copy(zbuf, acc.at[pl.ds(s * ROWS_T + m * CP, CP)])
    plsc.subcore_barrier()

    @pl.loop(0, NCHUNK)
    def _chunk(j):
      pltpu.async_copy(ast_hbm.at[nidx_v.at[j]], rows, sem).wait()
      pltpu.sync_copy(rows, acc.at[segv.at[j]], add=True)

    plsc.subcore_barrier()

    for m in range(NCP):
      r0 = s * ROWS_T + m * CP
      pltpu.sync_copy(acc.at[pl.ds(r0, CP)], zbuf)
      pltpu.sync_copy(zbuf, b_out.at[c, pl.ds(r0, CP)])

  f = pl.kernel(
      body,
      out_type=(
          jax.ShapeDtypeStruct((NW, SYM_W, D), jnp.float32),
          jax.ShapeDtypeStruct((NC, N_SYM, D), jnp.float32),
      ),
      mesh=mesh,
      scratch_types=(
          pltpu.VMEM((NCHUNK, CHUNK), jnp.int32),
          pltpu.VMEM((NCHUNK, CHUNK), jnp.int32),
          pltpu.VMEM((SYM_CHUNKS, CHUNK), jnp.int32),
          pltpu.VMEM((CHUNK, D), jnp.float32),
          pltpu.VMEM((CP, D), jnp.float32),
          pltpu.VMEM_SHARED((N_SYM, D), jnp.float32),
          pltpu.SemaphoreType.DMA,
      ),
  )
  return f(ident_tab, sym_idx, ast_tab, node_idx, seg_idx)


BLK = 1000


def _tc_combine(a, b_partial, w1, w2):
  def body(a_ref, b_ref, w1_ref, w2_ref, o_ref):
    acc = jnp.dot(a_ref[...], w1_ref[...],
                  preferred_element_type=jnp.float32,
                  precision=lax.Precision.HIGHEST)
    acc = acc + jnp.dot(b_ref[0] + b_ref[1], w2_ref[...],
                        preferred_element_type=jnp.float32,
                        precision=lax.Precision.HIGHEST)
    o_ref[...] = jnp.maximum(acc, 0.0)

  return pl.pallas_call(
      body,
      grid=(N_SYM // BLK,),
      in_specs=[
          pl.BlockSpec((BLK, D), lambda i: (i, 0)),
          pl.BlockSpec((NC, BLK, D), lambda i: (0, i, 0)),
          pl.BlockSpec((D, D), lambda i: (0, 0)),
          pl.BlockSpec((D, D), lambda i: (0, 0)),
      ],
      out_specs=pl.BlockSpec((BLK, D), lambda i: (i, 0)),
      out_shape=jax.ShapeDtypeStruct((N_SYM, D), jnp.float32),
  )(a, b_partial, w1, w2)


def kernel(encoded_identifiers, symbols_identifier_indices, encoded_ast_nodes,
           ast_nodes_with_symbol_leaf_nodes_indices,
           ast_nodes_with_symbol_leaf_symbol_idx, W):
  sym_idx = symbols_identifier_indices.astype(jnp.int32)
  sym_idx = jnp.concatenate(
      [sym_idx, jnp.zeros((SYM_PAD - N_SYM,), jnp.int32)]
  ).reshape(NW, SYM_CHUNKS, CHUNK)
  node_idx = ast_nodes_with_symbol_leaf_nodes_indices.astype(jnp.int32)
  node_idx = node_idx.reshape(NW, NCHUNK, CHUNK)
  seg_idx = ast_nodes_with_symbol_leaf_symbol_idx.astype(jnp.int32)
  seg_idx = seg_idx.reshape(NW, NCHUNK, CHUNK)

  a_gath, b_partial = _sc_gather_segsum(
      encoded_identifiers, sym_idx, encoded_ast_nodes, node_idx, seg_idx)
  a = a_gath.reshape(SYM_PAD, D)[:N_SYM]
  return _tc_combine(a, b_partial, W[:D], W[D:])


# speedup vs baseline: 6.7195x; 6.7195x over previous
import jax
import jax.numpy as jnp
from jax import lax
from jax.experimental import pallas as pl
from jax.experimental.pallas import tpu as pltpu
from jax.experimental.pallas import tpu_sc as plsc

N_IDENT = 10000
N_SYM = 10000
N_AST = 100000
N_OCC = 320000
D = 128

NC, NS = 2, 16
NW = NC * NS
OCC_W = N_OCC // NW
CHUNK = 80
NCHUNK = OCC_W // CHUNK
SYM_PAD = 10240
SYM_W = SYM_PAD // NW
SYM_CHUNKS = SYM_W // CHUNK
SEG_PAD = 10240
ROWS_T = SEG_PAD // NS
CP = 32
NCP = ROWS_T // CP


def _sc_gather_segsum(ident_tab, sym_idx, ast_tab, node_idx, seg_idx):
  mesh = plsc.VectorSubcoreMesh(
      core_axis_name="c", subcore_axis_name="s", num_cores=NC, num_subcores=NS)

  def body(ident_hbm, sym_hbm, ast_hbm, nidx_hbm, sidx_hbm, a_out, b_out,
           nidx_v, segv, symv, rows, zbuf, acc, sem):
    c = lax.axis_index("c")
    s = lax.axis_index("s")
    wid = s * NC + c

    pltpu.sync_copy(nidx_hbm.at[wid], nidx_v)
    pltpu.sync_copy(sidx_hbm.at[wid], segv)
    pltpu.sync_copy(sym_hbm.at[wid], symv)

    for k in range(SYM_CHUNKS):
      pltpu.async_copy(ident_hbm.at[symv.at[k]], rows, sem).wait()
      pltpu.sync_copy(rows, a_out.at[wid, pl.ds(k * CHUNK, CHUNK)])

    zero = jnp.zeros((16,), jnp.float32)

    @pl.loop(0, CP)
    def _zero_rows(i):
      for j in range(D // 16):
        zbuf[i, pl.ds(j * 16, 16)] = zero

    for m in range(NCP):
      pltpu.sync_copy(zbuf, acc.at[pl.ds(s * ROWS_T + m * CP, CP)])
    plsc.subcore_barrier()

    @pl.loop(0, NCHUNK)
    def _chunk(j):
      pltpu.async_copy(ast_hbm.at[nidx_v.at[j]], rows, sem).wait()
      pltpu.sync_copy(rows, acc.at[segv.at[j]], add=True)

    plsc.subcore_barrier()

    for m in range(NCP):
      r0 = s * ROWS_T + m * CP
      pltpu.sync_copy(acc.at[pl.ds(r0, CP)], zbuf)
      pltpu.sync_copy(zbuf, b_out.at[c, pl.ds(r0, CP)])

  f = pl.kernel(
      body,
      out_type=(
          jax.ShapeDtypeStruct((NW, SYM_W, D), jnp.float32),
          jax.ShapeDtypeStruct((NC, SEG_PAD, D), jnp.float32),
      ),
      mesh=mesh,
      scratch_types=(
          pltpu.VMEM((NCHUNK, CHUNK), jnp.int32),
          pltpu.VMEM((NCHUNK, CHUNK), jnp.int32),
          pltpu.VMEM((SYM_CHUNKS, CHUNK), jnp.int32),
          pltpu.VMEM((CHUNK, D), jnp.float32),
          pltpu.VMEM((CP, D), jnp.float32),
          pltpu.VMEM_SHARED((SEG_PAD, D), jnp.float32),
          pltpu.SemaphoreType.DMA,
      ),
  )
  return f(ident_tab, sym_idx, ast_tab, node_idx, seg_idx)


BLK = 1000


def _tc_combine(a, b_partial, w1, w2):
  def body(a_ref, b_ref, w1_ref, w2_ref, o_ref):
    acc = jnp.dot(a_ref[...], w1_ref[...],
                  preferred_element_type=jnp.float32,
                  precision=lax.Precision.HIGHEST)
    acc = acc + jnp.dot(b_ref[0] + b_ref[1], w2_ref[...],
                        preferred_element_type=jnp.float32,
                        precision=lax.Precision.HIGHEST)
    o_ref[...] = jnp.maximum(acc, 0.0)

  return pl.pallas_call(
      body,
      grid=(N_SYM // BLK,),
      in_specs=[
          pl.BlockSpec((BLK, D), lambda i: (i, 0)),
          pl.BlockSpec((NC, BLK, D), lambda i: (0, i, 0)),
          pl.BlockSpec((D, D), lambda i: (0, 0)),
          pl.BlockSpec((D, D), lambda i: (0, 0)),
      ],
      out_specs=pl.BlockSpec((BLK, D), lambda i: (i, 0)),
      out_shape=jax.ShapeDtypeStruct((N_SYM, D), jnp.float32),
  )(a, b_partial, w1, w2)


def kernel(encoded_identifiers, symbols_identifier_indices, encoded_ast_nodes,
           ast_nodes_with_symbol_leaf_nodes_indices,
           ast_nodes_with_symbol_leaf_symbol_idx, W):
  sym_idx = symbols_identifier_indices.astype(jnp.int32)
  sym_idx = jnp.concatenate(
      [sym_idx, jnp.zeros((SYM_PAD - N_SYM,), jnp.int32)]
  ).reshape(NW, SYM_CHUNKS, CHUNK)
  node_idx = ast_nodes_with_symbol_leaf_nodes_indices.astype(jnp.int32)
  node_idx = node_idx.reshape(NW, NCHUNK, CHUNK)
  seg_idx = ast_nodes_with_symbol_leaf_symbol_idx.astype(jnp.int32)
  seg_idx = seg_idx.reshape(NW, NCHUNK, CHUNK)

  a_gath, b_partial = _sc_gather_segsum(
      encoded_identifiers, sym_idx, encoded_ast_nodes, node_idx, seg_idx)
  a = a_gath.reshape(SYM_PAD, D)[:N_SYM]
  return _tc_combine(a, b_partial[:, :N_SYM], W[:D], W[D:])
